# baseline (speedup 1.0000x reference)
import functools

import jax
import jax.numpy as jnp
from jax import lax
from jax.experimental import pallas as pl
from jax.experimental.pallas import tpu as pltpu
from jax.experimental.pallas import tpu_sc as plsc

_NC = 2
_NS = 16
_NW = _NC * _NS
_L = 16
_TCOLS = 640


def _table_body(wemb_ref, whf_ref, bh_ref, t_ref):
    h_rows = t_ref.shape[0]
    d = wemb_ref.shape[1]
    wh = jnp.concatenate(
        [whf_ref[0:1, i * d:(i + 1) * d] for i in range(h_rows)], axis=0
    )
    dn = (((1,), (1,)), ((), ()))
    t = lax.dot_general(
        wh, wemb_ref[...], dimension_numbers=dn,
        preferred_element_type=jnp.float32,
    )
    t_ref[:, 0:512] = t + bh_ref[...].reshape(h_rows, 1)
    c = jnp.sum(wh, axis=1, keepdims=True)
    t_ref[:, 512:_TCOLS] = jnp.broadcast_to(c, (h_rows, _TCOLS - 512))


def _make_sc_combine(B, S, h_rows):
    chunk = B * S // _NS
    per_row = S // chunk
    n_it = chunk // _L
    mesh = plsc.VectorSubcoreMesh(
        core_axis_name="c", subcore_axis_name="s", num_cores=1
    )

    @functools.partial(
        pl.kernel,
        mesh=mesh,
        compiler_params=pltpu.CompilerParams(
            needs_layout_passes=False,
            disable_bounds_checks=True,
            disable_semaphore_checks=True,
        ),
        out_type=jax.ShapeDtypeStruct((B * S,), jnp.float32),
        scratch_types=[
            pltpu.VMEM((chunk,), jnp.int32),
            pltpu.VMEM((chunk,), jnp.int32),
            pltpu.VMEM((chunk,), jnp.float32),
            pltpu.VMEM((chunk,), jnp.float32),
            pltpu.VMEM((h_rows, _TCOLS), jnp.float32),
            pltpu.VMEM((chunk,), jnp.float32),
            pltpu.SemaphoreType.DMA,
        ],
    )
    def sc_combine(sel_hbm, tsk_hbm, val_hbm, msk_hbm, t_hbm, out_hbm,
                   sel_v, tsk_v, val_v, msk_v, t_v, out_v, sem):
        wid = lax.axis_index("s")
        b = wid // per_row
        s0 = (wid % per_row) * chunk
        cps = [
            pltpu.async_copy(t_hbm, t_v, sem),
            pltpu.async_copy(sel_hbm.at[b, pl.ds(s0, chunk)], sel_v, sem),
            pltpu.async_copy(tsk_hbm.at[b, pl.ds(s0, chunk)], tsk_v, sem),
            pltpu.async_copy(val_hbm.at[b, pl.ds(s0, chunk)], val_v, sem),
            pltpu.async_copy(msk_hbm.at[b, pl.ds(s0, chunk)], msk_v, sem),
        ]
        for cp in cps:
            cp.wait()

        c_col = jnp.full((_L,), 512, jnp.int32)

        def it(i, carry):
            ds = pl.ds(i * _L, _L)
            tsk = tsk_v[ds]
            tval = plsc.load_gather(t_v, [tsk, sel_v[ds]])
            cval = plsc.load_gather(t_v, [tsk, c_col])
            out_v[ds] = tval + val_v[ds] * msk_v[ds] * cval
            return carry

        lax.fori_loop(0, n_it, it, 0)
        pltpu.sync_copy(out_v, out_hbm.at[pl.ds(wid * chunk, chunk)])

    return sc_combine


def kernel(selfies, tasks, values, property_mask, W_emb, W_heads, b_heads):
    B, S = selfies.shape
    V, D = W_emb.shape
    H, _, O = W_heads.shape

    t_tab = pl.pallas_call(
        _table_body,
        out_shape=jax.ShapeDtypeStruct((H, _TCOLS), jnp.float32),
    )(W_emb, W_heads.reshape(1, H * D), b_heads.reshape(1, H))

    out_flat = _make_sc_combine(B, S, H)(
        selfies.astype(jnp.int32),
        tasks.astype(jnp.int32),
        values,
        property_mask,
        t_tab,
    )
    return out_flat.reshape(B, S, O)


# speedup vs baseline: 1.0010x; 1.0010x over previous
import functools

import jax
import jax.numpy as jnp
from jax import lax
from jax.experimental import pallas as pl
from jax.experimental.pallas import tpu as pltpu
from jax.experimental.pallas import tpu_sc as plsc

_NS = 16
_L = 16
_TCOLS = 640


def _table_body(wemb_ref, whf_ref, bh_ref, t_ref):
    h_rows = t_ref.shape[0]
    d = wemb_ref.shape[1]
    wh = jnp.concatenate(
        [whf_ref[0:1, i * d:(i + 1) * d] for i in range(h_rows)], axis=0
    )
    dn = (((1,), (1,)), ((), ()))
    t = lax.dot_general(
        wh, wemb_ref[...], dimension_numbers=dn,
        preferred_element_type=jnp.float32,
    )
    t_ref[:, 0:512] = t + bh_ref[...].reshape(h_rows, 1)
    c = jnp.sum(wh, axis=1, keepdims=True)
    t_ref[:, 512:_TCOLS] = jnp.broadcast_to(c, (h_rows, _TCOLS - 512))


def _make_sc_combine(B, S, h_rows):
    chunk = B * S // _NS
    per_row = S // chunk
    n_it = chunk // _L
    mesh = plsc.VectorSubcoreMesh(
        core_axis_name="c", subcore_axis_name="s", num_cores=1
    )

    @functools.partial(
        pl.kernel,
        mesh=mesh,
        compiler_params=pltpu.CompilerParams(needs_layout_passes=False),
        out_type=jax.ShapeDtypeStruct((B * S,), jnp.float32),
        scratch_types=[
            pltpu.VMEM((chunk,), jnp.int32),
            pltpu.VMEM((chunk,), jnp.int32),
            pltpu.VMEM((chunk,), jnp.float32),
            pltpu.VMEM((chunk,), jnp.float32),
            pltpu.VMEM((h_rows, _TCOLS), jnp.float32),
            pltpu.VMEM((chunk,), jnp.float32),
            pltpu.SemaphoreType.DMA,
        ],
    )
    def sc_combine(sel_hbm, tsk_hbm, val_hbm, msk_hbm, t_hbm, out_hbm,
                   sel_v, tsk_v, val_v, msk_v, t_v, out_v, sem):
        wid = lax.axis_index("s")
        b = wid // per_row
        s0 = (wid % per_row) * chunk
        cps = [
            pltpu.async_copy(t_hbm, t_v, sem),
            pltpu.async_copy(sel_hbm.at[b, pl.ds(s0, chunk)], sel_v, sem),
            pltpu.async_copy(tsk_hbm.at[b, pl.ds(s0, chunk)], tsk_v, sem),
            pltpu.async_copy(val_hbm.at[b, pl.ds(s0, chunk)], val_v, sem),
            pltpu.async_copy(msk_hbm.at[b, pl.ds(s0, chunk)], msk_v, sem),
        ]
        for cp in cps:
            cp.wait()

        c_col = jnp.full((_L,), 512, jnp.int32)

        def it(i, carry):
            ds = pl.ds(i * _L, _L)
            tsk = tsk_v[ds]
            tval = plsc.load_gather(t_v, [tsk, sel_v[ds]])
            cval = plsc.load_gather(t_v, [tsk, c_col])
            out_v[ds] = tval + val_v[ds] * msk_v[ds] * cval
            return carry

        lax.fori_loop(0, n_it, it, 0)
        pltpu.sync_copy(out_v, out_hbm.at[pl.ds(wid * chunk, chunk)])

    return sc_combine


def kernel(selfies, tasks, values, property_mask, W_emb, W_heads, b_heads):
    B, S = selfies.shape
    V, D = W_emb.shape
    H, _, O = W_heads.shape

    t_tab = pl.pallas_call(
        _table_body,
        out_shape=jax.ShapeDtypeStruct((H, _TCOLS), jnp.float32),
    )(W_emb, W_heads.reshape(1, H * D), b_heads.reshape(1, H))

    out_flat = _make_sc_combine(B, S, H)(
        selfies.astype(jnp.int32),
        tasks.astype(jnp.int32),
        values,
        property_mask,
        t_tab,
    )
    return out_flat.reshape(B, S, O)
